# Initial kernel scaffold; baseline (speedup 1.0000x reference)
import jax
import jax.numpy as jnp
from jax.experimental import pallas as pl


def kernel(x, table):
    raise NotImplementedError("write your pallas kernel here")


# speedup vs baseline: 1.0289x; 1.0289x over previous
import functools

import jax
import jax.numpy as jnp
from jax import lax
from jax.experimental import pallas as pl
from jax.experimental.pallas import tpu as pltpu
from jax.experimental.pallas import tpu_sc as plsc

_BB = 1024
_KB = 1024
_SEG = (2736, 5472)


def _argmin_body(nk_total, x_ref, t_ref, ix_ref, sm_ref, si_ref, gm_ref, gi_ref):
    j = pl.program_id(1)

    @pl.when(j == 0)
    def _init():
        sm_ref[...] = jnp.full(sm_ref.shape, jnp.inf, sm_ref.dtype)
        si_ref[...] = jnp.zeros(si_ref.shape, si_ref.dtype)
        gm_ref[...] = jnp.full(gm_ref.shape, jnp.inf, gm_ref.dtype)
        gi_ref[...] = jnp.zeros(gi_ref.shape, gi_ref.dtype)

    scores = lax.dot_general(
        x_ref[...].astype(jnp.bfloat16), t_ref[...].astype(jnp.bfloat16),
        dimension_numbers=(((1,), (1,)), ((), ())),
        preferred_element_type=jnp.float32)
    kb = scores.shape[1]
    ktot = nk_total * kb
    col_local = lax.broadcasted_iota(jnp.int32, scores.shape, 1)
    col = col_local + j * kb

    def piece_minarg(lo, hi):
        if lo == 0 and hi == kb:
            s = scores
        else:
            mask = (col_local >= lo) & (col_local < hi)
            s = jnp.where(mask, scores, jnp.inf)
        lm = jnp.min(s, axis=1, keepdims=True)
        la = jnp.min(jnp.where(s == lm, col, ktot), axis=1, keepdims=True)
        return lm, la

    def merge(lm, la):
        upd = lm < sm_ref[...]
        sm_ref[...] = jnp.where(upd, lm, sm_ref[...])
        si_ref[...] = jnp.where(upd, la, si_ref[...])

    def finalize():
        upd = sm_ref[...] < gm_ref[...]
        gm = jnp.where(upd, sm_ref[...], gm_ref[...])
        gm_ref[...] = gm.astype(jnp.bfloat16).astype(jnp.float32)
        gi_ref[...] = jnp.where(upd, si_ref[...], gi_ref[...])
        sm_ref[...] = jnp.full(sm_ref.shape, jnp.inf, sm_ref.dtype)
        si_ref[...] = jnp.zeros(si_ref.shape, si_ref.dtype)

    cut_blocks = {b // kb: b % kb for b in _SEG}
    assert all(off != 0 for off in cut_blocks.values())

    is_plain = jnp.bool_(True)
    for jj in cut_blocks:
        is_plain &= j != jj

    @pl.when(is_plain)
    def _plain():
        merge(*piece_minarg(0, kb))

    for jj, off in sorted(cut_blocks.items()):
        @pl.when(j == jj)
        def _split(off=off):
            merge(*piece_minarg(0, off))
            finalize()
            merge(*piece_minarg(off, kb))

    @pl.when(j == nk_total - 1)
    def _last():
        finalize()
        ix_ref[...] = gi_ref[:, 0]


def _scores_argmin(x, table):
    b, d = x.shape
    kk = table.shape[0]
    nk = kk // _KB
    grid = (b // _BB, nk)
    return pl.pallas_call(
        functools.partial(_argmin_body, nk),
        grid=grid,
        in_specs=[
            pl.BlockSpec((_BB, d), lambda i, j: (i, 0)),
            pl.BlockSpec((_KB, d), lambda i, j: (j, 0)),
        ],
        out_specs=pl.BlockSpec((_BB,), lambda i, j: (i,)),
        out_shape=jax.ShapeDtypeStruct((b,), jnp.int32),
        scratch_shapes=[
            pltpu.VMEM((_BB, 1), jnp.float32),
            pltpu.VMEM((_BB, 1), jnp.int32),
            pltpu.VMEM((_BB, 1), jnp.float32),
            pltpu.VMEM((_BB, 1), jnp.int32),
        ],
    )(x, table)


def _gather_rows(table, ix):
    kk, d = table.shape
    b = ix.shape[0]
    info = plsc.get_sparse_core_info()
    nw = info.num_cores * info.num_subcores
    bpw = b // nw
    ch = 128
    nch = bpw // ch
    mesh = plsc.VectorSubcoreMesh(core_axis_name="c", subcore_axis_name="s")

    @functools.partial(
        pl.kernel, mesh=mesh,
        out_type=jax.ShapeDtypeStruct((b, d), jnp.float32),
        scratch_types=[
            pltpu.VMEM((ch,), jnp.int32),
            pltpu.VMEM((ch, d), jnp.float32),
            pltpu.SemaphoreType.DMA,
        ],
    )
    def gk(table_hbm, idx_hbm, out_hbm, idx_v, rows_v, sem):
        wid = lax.axis_index("s") * info.num_cores + lax.axis_index("c")
        for c in range(nch):
            base = wid * bpw + c * ch
            pltpu.sync_copy(idx_hbm.at[pl.ds(base, ch)], idx_v)
            pltpu.async_copy(table_hbm.at[idx_v], rows_v, sem).wait()
            pltpu.sync_copy(rows_v, out_hbm.at[pl.ds(base, ch)])

    return gk(table, ix)


def kernel(x, table):
    ix = _scores_argmin(x, table)
    out = _gather_rows(table, ix)
    return out.reshape(x.shape)


# speedup vs baseline: 1.2182x; 1.1839x over previous
import functools

import jax
import jax.numpy as jnp
from jax import lax
from jax.experimental import pallas as pl
from jax.experimental.pallas import tpu as pltpu
from jax.experimental.pallas import tpu_sc as plsc

_BB = 1024
_KB = 1024
_SEG = (2736, 5472)


def _argmin_body(nk_total, x_ref, t_ref, ix_ref, sm_ref, si_ref, gm_ref, gi_ref):
    j = pl.program_id(1)

    @pl.when(j == 0)
    def _init():
        sm_ref[...] = jnp.full(sm_ref.shape, jnp.inf, sm_ref.dtype)
        si_ref[...] = jnp.zeros(si_ref.shape, si_ref.dtype)
        gm_ref[...] = jnp.full(gm_ref.shape, jnp.inf, gm_ref.dtype)
        gi_ref[...] = jnp.zeros(gi_ref.shape, gi_ref.dtype)

    scores = lax.dot_general(
        t_ref[...].astype(jnp.bfloat16), x_ref[...].astype(jnp.bfloat16),
        dimension_numbers=(((1,), (1,)), ((), ())),
        preferred_element_type=jnp.float32)
    kb = scores.shape[0]
    ktot = nk_total * kb
    row_local = lax.broadcasted_iota(jnp.int32, scores.shape, 0)
    row = row_local + j * kb

    def piece_minarg(lo, hi):
        if lo == 0 and hi == kb:
            s = scores
        else:
            mask = (row_local >= lo) & (row_local < hi)
            s = jnp.where(mask, scores, jnp.inf)
        lm = jnp.min(s, axis=0, keepdims=True)
        la = jnp.min(jnp.where(s == lm, row, ktot), axis=0, keepdims=True)
        return lm, la

    def merge(lm, la):
        upd = lm < sm_ref[...]
        sm_ref[...] = jnp.where(upd, lm, sm_ref[...])
        si_ref[...] = jnp.where(upd, la, si_ref[...])

    def finalize():
        upd = sm_ref[...] < gm_ref[...]
        gm = jnp.where(upd, sm_ref[...], gm_ref[...])
        gm_ref[...] = gm.astype(jnp.bfloat16).astype(jnp.float32)
        gi_ref[...] = jnp.where(upd, si_ref[...], gi_ref[...])
        sm_ref[...] = jnp.full(sm_ref.shape, jnp.inf, sm_ref.dtype)
        si_ref[...] = jnp.zeros(si_ref.shape, si_ref.dtype)

    cut_blocks = {b // kb: b % kb for b in _SEG}
    assert all(off != 0 for off in cut_blocks.values())

    is_plain = jnp.bool_(True)
    for jj in cut_blocks:
        is_plain &= j != jj

    @pl.when(is_plain)
    def _plain():
        merge(*piece_minarg(0, kb))

    for jj, off in sorted(cut_blocks.items()):
        @pl.when(j == jj)
        def _split(off=off):
            merge(*piece_minarg(0, off))
            finalize()
            merge(*piece_minarg(off, kb))

    @pl.when(j == nk_total - 1)
    def _last():
        finalize()
        ix_ref[...] = gi_ref[0, :]


def _scores_argmin(x, table):
    b, d = x.shape
    kk = table.shape[0]
    nk = kk // _KB
    grid = (b // _BB, nk)
    return pl.pallas_call(
        functools.partial(_argmin_body, nk),
        grid=grid,
        in_specs=[
            pl.BlockSpec((_BB, d), lambda i, j: (i, 0)),
            pl.BlockSpec((_KB, d), lambda i, j: (j, 0)),
        ],
        out_specs=pl.BlockSpec((_BB,), lambda i, j: (i,)),
        out_shape=jax.ShapeDtypeStruct((b,), jnp.int32),
        scratch_shapes=[
            pltpu.VMEM((1, _BB), jnp.float32),
            pltpu.VMEM((1, _BB), jnp.int32),
            pltpu.VMEM((1, _BB), jnp.float32),
            pltpu.VMEM((1, _BB), jnp.int32),
        ],
    )(x, table)


def _gather_rows(table, ix):
    kk, d = table.shape
    b = ix.shape[0]
    info = plsc.get_sparse_core_info()
    nw = info.num_cores * info.num_subcores
    bpw = b // nw
    ch = 128
    nch = bpw // ch
    mesh = plsc.VectorSubcoreMesh(core_axis_name="c", subcore_axis_name="s")

    @functools.partial(
        pl.kernel, mesh=mesh,
        out_type=jax.ShapeDtypeStruct((b, d), jnp.float32),
        scratch_types=[
            pltpu.VMEM((ch,), jnp.int32),
            pltpu.VMEM((ch, d), jnp.float32),
            pltpu.SemaphoreType.DMA,
        ],
    )
    def gk(table_hbm, idx_hbm, out_hbm, idx_v, rows_v, sem):
        wid = lax.axis_index("s") * info.num_cores + lax.axis_index("c")
        for c in range(nch):
            base = wid * bpw + c * ch
            pltpu.sync_copy(idx_hbm.at[pl.ds(base, ch)], idx_v)
            pltpu.async_copy(table_hbm.at[idx_v], rows_v, sem).wait()
            pltpu.sync_copy(rows_v, out_hbm.at[pl.ds(base, ch)])

    return gk(table, ix)


def kernel(x, table):
    ix = _scores_argmin(x, table)
    out = _gather_rows(table, ix)
    return out.reshape(x.shape)


# speedup vs baseline: 1.4058x; 1.1540x over previous
import functools

import jax
import jax.numpy as jnp
from jax import lax
from jax.experimental import pallas as pl
from jax.experimental.pallas import tpu as pltpu
from jax.experimental.pallas import tpu_sc as plsc

_BB = 1024
_KB = 1024
_SEG = (2736, 5472)


def _argmin_body(nk_total, x_ref, t_ref, ix_ref, sm_ref, si_ref, gm_ref, gi_ref):
    j = pl.program_id(1)

    @pl.when(j == 0)
    def _init():
        sm_ref[...] = jnp.full(sm_ref.shape, jnp.inf, sm_ref.dtype)
        si_ref[...] = jnp.zeros(si_ref.shape, si_ref.dtype)
        gm_ref[...] = jnp.full(gm_ref.shape, jnp.inf, gm_ref.dtype)
        gi_ref[...] = jnp.zeros(gi_ref.shape, gi_ref.dtype)

    scores = lax.dot_general(
        t_ref[...].astype(jnp.bfloat16), x_ref[...].astype(jnp.bfloat16),
        dimension_numbers=(((1,), (1,)), ((), ())),
        preferred_element_type=jnp.float32)
    kb, bb = scores.shape
    ktot = nk_total * kb
    s3 = scores.reshape(kb // 8, 8, bb)
    siota = lax.broadcasted_iota(jnp.int32, (1, 8, bb), 1)[0]

    def scan_rows(rows):
        half = len(rows) // 2
        chains = []
        for part in (rows[:half], rows[half:]):
            m = s3[part[0]]
            ri = jnp.full((8, bb), part[0], jnp.int32)
            for r in part[1:]:
                v = s3[r]
                lt = v < m
                m = jnp.minimum(m, v)
                ri = jnp.where(lt, r, ri)
            chains.append((m, ri))
        (m0, r0), (m1, r1) = chains
        lt = m1 < m0
        return jnp.minimum(m0, m1), jnp.where(lt, r1, r0)

    def piece_minarg(lo, hi):
        m, ri = scan_rows(list(range(lo // 8, hi // 8)))
        kfull = ri * 8 + siota + j * kb
        lm = jnp.min(m, axis=0, keepdims=True)
        la = jnp.min(jnp.where(m == lm, kfull, ktot), axis=0, keepdims=True)
        return lm, la

    def merge(lm, la):
        upd = lm < sm_ref[...]
        sm_ref[...] = jnp.where(upd, lm, sm_ref[...])
        si_ref[...] = jnp.where(upd, la, si_ref[...])

    def finalize():
        upd = sm_ref[...] < gm_ref[...]
        gm = jnp.where(upd, sm_ref[...], gm_ref[...])
        gm_ref[...] = gm.astype(jnp.bfloat16).astype(jnp.float32)
        gi_ref[...] = jnp.where(upd, si_ref[...], gi_ref[...])
        sm_ref[...] = jnp.full(sm_ref.shape, jnp.inf, sm_ref.dtype)
        si_ref[...] = jnp.zeros(si_ref.shape, si_ref.dtype)

    cut_blocks = {b // kb: b % kb for b in _SEG}
    assert all(off != 0 for off in cut_blocks.values())

    is_plain = jnp.bool_(True)
    for jj in cut_blocks:
        is_plain &= j != jj

    @pl.when(is_plain)
    def _plain():
        merge(*piece_minarg(0, kb))

    for jj, off in sorted(cut_blocks.items()):
        @pl.when(j == jj)
        def _split(off=off):
            merge(*piece_minarg(0, off))
            finalize()
            merge(*piece_minarg(off, kb))

    @pl.when(j == nk_total - 1)
    def _last():
        finalize()
        ix_ref[...] = gi_ref[0, :]


def _scores_argmin(x, table):
    b, d = x.shape
    kk = table.shape[0]
    nk = kk // _KB
    grid = (b // _BB, nk)
    return pl.pallas_call(
        functools.partial(_argmin_body, nk),
        grid=grid,
        in_specs=[
            pl.BlockSpec((_BB, d), lambda i, j: (i, 0)),
            pl.BlockSpec((_KB, d), lambda i, j: (j, 0)),
        ],
        out_specs=pl.BlockSpec((_BB,), lambda i, j: (i,)),
        out_shape=jax.ShapeDtypeStruct((b,), jnp.int32),
        scratch_shapes=[
            pltpu.VMEM((1, _BB), jnp.float32),
            pltpu.VMEM((1, _BB), jnp.int32),
            pltpu.VMEM((1, _BB), jnp.float32),
            pltpu.VMEM((1, _BB), jnp.int32),
        ],
    )(x, table)


def _gather_rows(table, ix):
    kk, d = table.shape
    b = ix.shape[0]
    info = plsc.get_sparse_core_info()
    nw = info.num_cores * info.num_subcores
    bpw = b // nw
    ch = 128
    nch = bpw // ch
    mesh = plsc.VectorSubcoreMesh(core_axis_name="c", subcore_axis_name="s")

    @functools.partial(
        pl.kernel, mesh=mesh,
        out_type=jax.ShapeDtypeStruct((b, d), jnp.float32),
        scratch_types=[
            pltpu.VMEM((ch,), jnp.int32),
            pltpu.VMEM((ch, d), jnp.float32),
            pltpu.SemaphoreType.DMA,
        ],
    )
    def gk(table_hbm, idx_hbm, out_hbm, idx_v, rows_v, sem):
        wid = lax.axis_index("s") * info.num_cores + lax.axis_index("c")
        for c in range(nch):
            base = wid * bpw + c * ch
            pltpu.sync_copy(idx_hbm.at[pl.ds(base, ch)], idx_v)
            pltpu.async_copy(table_hbm.at[idx_v], rows_v, sem).wait()
            pltpu.sync_copy(rows_v, out_hbm.at[pl.ds(base, ch)])

    return gk(table, ix)


def kernel(x, table):
    ix = _scores_argmin(x, table)
    out = _gather_rows(table, ix)
    return out.reshape(x.shape)


# speedup vs baseline: 1.4651x; 1.0422x over previous
import functools

import jax
import jax.numpy as jnp
from jax import lax
from jax.experimental import pallas as pl
from jax.experimental.pallas import tpu as pltpu
from jax.experimental.pallas import tpu_sc as plsc

_BB = 1024
_KB = 1024
_SEG = (2736, 5472)


def _argmin_body(nk_total, x_ref, t_ref, ix_ref, sm_ref, si_ref, gm_ref, gi_ref):
    j = pl.program_id(1)

    @pl.when(j == 0)
    def _init():
        sm_ref[...] = jnp.full(sm_ref.shape, jnp.inf, sm_ref.dtype)
        si_ref[...] = jnp.zeros(si_ref.shape, si_ref.dtype)
        gm_ref[...] = jnp.full(gm_ref.shape, jnp.inf, gm_ref.dtype)
        gi_ref[...] = jnp.zeros(gi_ref.shape, gi_ref.dtype)

    scores = lax.dot_general(
        t_ref[...].astype(jnp.bfloat16), x_ref[...].astype(jnp.bfloat16),
        dimension_numbers=(((1,), (1,)), ((), ())),
        preferred_element_type=jnp.float32)
    kb, bb = scores.shape
    ktot = nk_total * kb
    s3 = scores.reshape(kb // 8, 8, bb)
    siota = lax.broadcasted_iota(jnp.int32, (1, 8, bb), 1)[0]

    def scan_rows(rows):
        m = s3[rows[0]]
        ri = jnp.full((8, bb), rows[0], jnp.int32)
        for r in rows[1:]:
            mn = jnp.minimum(m, s3[r])
            ri = jnp.where(mn < m, r, ri)
            m = mn
        return m, ri

    def piece_minarg(lo, hi):
        m, ri = scan_rows(list(range(lo // 8, hi // 8)))
        kfull = ri * 8 + siota + j * kb
        lm = jnp.min(m, axis=0, keepdims=True)
        la = jnp.min(jnp.where(m == lm, kfull, ktot), axis=0, keepdims=True)
        return lm, la

    def merge(lm, la):
        upd = lm < sm_ref[...]
        sm_ref[...] = jnp.where(upd, lm, sm_ref[...])
        si_ref[...] = jnp.where(upd, la, si_ref[...])

    def finalize():
        upd = sm_ref[...] < gm_ref[...]
        gm = jnp.where(upd, sm_ref[...], gm_ref[...])
        gm_ref[...] = gm.astype(jnp.bfloat16).astype(jnp.float32)
        gi_ref[...] = jnp.where(upd, si_ref[...], gi_ref[...])
        sm_ref[...] = jnp.full(sm_ref.shape, jnp.inf, sm_ref.dtype)
        si_ref[...] = jnp.zeros(si_ref.shape, si_ref.dtype)

    cut_blocks = {b // kb: b % kb for b in _SEG}
    assert all(off != 0 for off in cut_blocks.values())

    is_plain = jnp.bool_(True)
    for jj in cut_blocks:
        is_plain &= j != jj

    @pl.when(is_plain)
    def _plain():
        merge(*piece_minarg(0, kb))

    for jj, off in sorted(cut_blocks.items()):
        @pl.when(j == jj)
        def _split(off=off):
            merge(*piece_minarg(0, off))
            finalize()
            merge(*piece_minarg(off, kb))

    @pl.when(j == nk_total - 1)
    def _last():
        finalize()
        ix_ref[...] = gi_ref[0, :]


def _scores_argmin(x, table):
    b, d = x.shape
    kk = table.shape[0]
    nk = kk // _KB
    grid = (b // _BB, nk)
    return pl.pallas_call(
        functools.partial(_argmin_body, nk),
        grid=grid,
        in_specs=[
            pl.BlockSpec((_BB, d), lambda i, j: (i, 0)),
            pl.BlockSpec((_KB, d), lambda i, j: (j, 0)),
        ],
        out_specs=pl.BlockSpec((_BB,), lambda i, j: (i,)),
        out_shape=jax.ShapeDtypeStruct((b,), jnp.int32),
        scratch_shapes=[
            pltpu.VMEM((1, _BB), jnp.float32),
            pltpu.VMEM((1, _BB), jnp.int32),
            pltpu.VMEM((1, _BB), jnp.float32),
            pltpu.VMEM((1, _BB), jnp.int32),
        ],
    )(x, table)


def _gather_rows(table, ix):
    kk, d = table.shape
    b = ix.shape[0]
    info = plsc.get_sparse_core_info()
    nw = info.num_cores * info.num_subcores
    bpw = b // nw
    ch = 128
    nch = bpw // ch
    mesh = plsc.VectorSubcoreMesh(core_axis_name="c", subcore_axis_name="s")

    @functools.partial(
        pl.kernel, mesh=mesh,
        out_type=jax.ShapeDtypeStruct((b, d), jnp.float32),
        scratch_types=[
            pltpu.VMEM((ch,), jnp.int32),
            pltpu.VMEM((ch, d), jnp.float32),
            pltpu.SemaphoreType.DMA,
        ],
    )
    def gk(table_hbm, idx_hbm, out_hbm, idx_v, rows_v, sem):
        wid = lax.axis_index("s") * info.num_cores + lax.axis_index("c")
        for c in range(nch):
            base = wid * bpw + c * ch
            pltpu.sync_copy(idx_hbm.at[pl.ds(base, ch)], idx_v)
            pltpu.async_copy(table_hbm.at[idx_v], rows_v, sem).wait()
            pltpu.sync_copy(rows_v, out_hbm.at[pl.ds(base, ch)])

    return gk(table, ix)


def kernel(x, table):
    ix = _scores_argmin(x, table)
    out = _gather_rows(table, ix)
    return out.reshape(x.shape)


# speedup vs baseline: 1.6291x; 1.1119x over previous
import functools

import jax
import jax.numpy as jnp
from jax import lax
from jax.experimental import pallas as pl
from jax.experimental.pallas import tpu as pltpu
from jax.experimental.pallas import tpu_sc as plsc

_BB = 1024
_KB = 2048
_SEG = (2736, 5472)


def _argmin_body(nk_total, x_ref, t_ref, ix_ref, sm_ref, si_ref, gm_ref, gi_ref):
    j = pl.program_id(1)

    @pl.when(j == 0)
    def _init():
        sm_ref[...] = jnp.full(sm_ref.shape, jnp.inf, sm_ref.dtype)
        si_ref[...] = jnp.zeros(si_ref.shape, si_ref.dtype)
        gm_ref[...] = jnp.full(gm_ref.shape, jnp.inf, gm_ref.dtype)
        gi_ref[...] = jnp.zeros(gi_ref.shape, gi_ref.dtype)

    scores = lax.dot_general(
        t_ref[...].astype(jnp.bfloat16), x_ref[...].astype(jnp.bfloat16),
        dimension_numbers=(((1,), (1,)), ((), ())),
        preferred_element_type=jnp.float32)
    kb, bb = scores.shape
    ktot = nk_total * kb
    s3 = scores.reshape(kb // 8, 8, bb)
    siota = lax.broadcasted_iota(jnp.int32, (1, 8, bb), 1)[0]

    def scan_rows(rows):
        m = s3[rows[0]]
        ri = jnp.full((8, bb), rows[0], jnp.int32)
        for r in rows[1:]:
            mn = jnp.minimum(m, s3[r])
            ri = jnp.where(mn < m, r, ri)
            m = mn
        return m, ri

    def piece_minarg(lo, hi):
        m, ri = scan_rows(list(range(lo // 8, hi // 8)))
        kfull = ri * 8 + siota + j * kb
        lm = jnp.min(m, axis=0, keepdims=True)
        la = jnp.min(jnp.where(m == lm, kfull, ktot), axis=0, keepdims=True)
        return lm, la

    def merge(lm, la):
        upd = lm < sm_ref[...]
        sm_ref[...] = jnp.where(upd, lm, sm_ref[...])
        si_ref[...] = jnp.where(upd, la, si_ref[...])

    def finalize():
        upd = sm_ref[...] < gm_ref[...]
        gm = jnp.where(upd, sm_ref[...], gm_ref[...])
        gm_ref[...] = gm.astype(jnp.bfloat16).astype(jnp.float32)
        gi_ref[...] = jnp.where(upd, si_ref[...], gi_ref[...])
        sm_ref[...] = jnp.full(sm_ref.shape, jnp.inf, sm_ref.dtype)
        si_ref[...] = jnp.zeros(si_ref.shape, si_ref.dtype)

    cut_blocks = {b // kb: b % kb for b in _SEG}
    assert all(off != 0 for off in cut_blocks.values())

    is_plain = jnp.bool_(True)
    for jj in cut_blocks:
        is_plain &= j != jj

    @pl.when(is_plain)
    def _plain():
        merge(*piece_minarg(0, kb))

    for jj, off in sorted(cut_blocks.items()):
        @pl.when(j == jj)
        def _split(off=off):
            merge(*piece_minarg(0, off))
            finalize()
            merge(*piece_minarg(off, kb))

    @pl.when(j == nk_total - 1)
    def _last():
        finalize()
        ix_ref[...] = gi_ref[0, :]


def _scores_argmin(x, table):
    b, d = x.shape
    kk = table.shape[0]
    nk = kk // _KB
    grid = (b // _BB, nk)
    return pl.pallas_call(
        functools.partial(_argmin_body, nk),
        grid=grid,
        in_specs=[
            pl.BlockSpec((_BB, d), lambda i, j: (i, 0)),
            pl.BlockSpec((_KB, d), lambda i, j: (j, 0)),
        ],
        out_specs=pl.BlockSpec((_BB,), lambda i, j: (i,)),
        out_shape=jax.ShapeDtypeStruct((b,), jnp.int32),
        scratch_shapes=[
            pltpu.VMEM((1, _BB), jnp.float32),
            pltpu.VMEM((1, _BB), jnp.int32),
            pltpu.VMEM((1, _BB), jnp.float32),
            pltpu.VMEM((1, _BB), jnp.int32),
        ],
    )(x, table)


def _gather_rows(table, ix):
    kk, d = table.shape
    b = ix.shape[0]
    info = plsc.get_sparse_core_info()
    nw = info.num_cores * info.num_subcores
    bpw = b // nw
    ch = 128
    nch = bpw // ch
    mesh = plsc.VectorSubcoreMesh(core_axis_name="c", subcore_axis_name="s")

    @functools.partial(
        pl.kernel, mesh=mesh,
        out_type=jax.ShapeDtypeStruct((b, d), jnp.float32),
        scratch_types=[
            pltpu.VMEM((ch,), jnp.int32),
            pltpu.VMEM((ch, d), jnp.float32),
            pltpu.SemaphoreType.DMA,
        ],
    )
    def gk(table_hbm, idx_hbm, out_hbm, idx_v, rows_v, sem):
        wid = lax.axis_index("s") * info.num_cores + lax.axis_index("c")
        for c in range(nch):
            base = wid * bpw + c * ch
            pltpu.sync_copy(idx_hbm.at[pl.ds(base, ch)], idx_v)
            pltpu.async_copy(table_hbm.at[idx_v], rows_v, sem).wait()
            pltpu.sync_copy(rows_v, out_hbm.at[pl.ds(base, ch)])

    return gk(table, ix)


def kernel(x, table):
    ix = _scores_argmin(x, table)
    out = _gather_rows(table, ix)
    return out.reshape(x.shape)


# speedup vs baseline: 1.7101x; 1.0497x over previous
import functools

import jax
import jax.numpy as jnp
from jax import lax
from jax.experimental import pallas as pl
from jax.experimental.pallas import tpu as pltpu
from jax.experimental.pallas import tpu_sc as plsc

_BB = 1024
_KB = 4096
_SEG = (2736, 5472)


def _argmin_body(nk_total, x_ref, t_ref, ix_ref, sm_ref, si_ref, gm_ref, gi_ref):
    j = pl.program_id(1)

    @pl.when(j == 0)
    def _init():
        sm_ref[...] = jnp.full(sm_ref.shape, jnp.inf, sm_ref.dtype)
        si_ref[...] = jnp.zeros(si_ref.shape, si_ref.dtype)
        gm_ref[...] = jnp.full(gm_ref.shape, jnp.inf, gm_ref.dtype)
        gi_ref[...] = jnp.zeros(gi_ref.shape, gi_ref.dtype)

    scores = lax.dot_general(
        t_ref[...].astype(jnp.bfloat16), x_ref[...].astype(jnp.bfloat16),
        dimension_numbers=(((1,), (1,)), ((), ())),
        preferred_element_type=jnp.float32)
    kb, bb = scores.shape
    ktot = nk_total * kb
    s3 = scores.reshape(kb // 8, 8, bb)
    siota = lax.broadcasted_iota(jnp.int32, (1, 8, bb), 1)[0]

    def scan_rows(rows):
        m = s3[rows[0]]
        ri = jnp.full((8, bb), rows[0], jnp.int32)
        for r in rows[1:]:
            mn = jnp.minimum(m, s3[r])
            ri = jnp.where(mn < m, r, ri)
            m = mn
        return m, ri

    def piece_minarg(lo, hi):
        m, ri = scan_rows(list(range(lo // 8, hi // 8)))
        kfull = ri * 8 + siota + j * kb
        lm = jnp.min(m, axis=0, keepdims=True)
        la = jnp.min(jnp.where(m == lm, kfull, ktot), axis=0, keepdims=True)
        return lm, la

    def merge(lm, la):
        upd = lm < sm_ref[...]
        sm_ref[...] = jnp.where(upd, lm, sm_ref[...])
        si_ref[...] = jnp.where(upd, la, si_ref[...])

    def finalize():
        upd = sm_ref[...] < gm_ref[...]
        gm = jnp.where(upd, sm_ref[...], gm_ref[...])
        gm_ref[...] = gm.astype(jnp.bfloat16).astype(jnp.float32)
        gi_ref[...] = jnp.where(upd, si_ref[...], gi_ref[...])
        sm_ref[...] = jnp.full(sm_ref.shape, jnp.inf, sm_ref.dtype)
        si_ref[...] = jnp.zeros(si_ref.shape, si_ref.dtype)

    cut_blocks = {b // kb: b % kb for b in _SEG}
    assert all(off != 0 for off in cut_blocks.values())

    is_plain = jnp.bool_(True)
    for jj in cut_blocks:
        is_plain &= j != jj

    @pl.when(is_plain)
    def _plain():
        merge(*piece_minarg(0, kb))

    for jj, off in sorted(cut_blocks.items()):
        @pl.when(j == jj)
        def _split(off=off):
            merge(*piece_minarg(0, off))
            finalize()
            merge(*piece_minarg(off, kb))

    @pl.when(j == nk_total - 1)
    def _last():
        finalize()
        ix_ref[...] = gi_ref[0, :]


def _scores_argmin(x, table):
    b, d = x.shape
    kk = table.shape[0]
    nk = kk // _KB
    grid = (b // _BB, nk)
    return pl.pallas_call(
        functools.partial(_argmin_body, nk),
        grid=grid,
        in_specs=[
            pl.BlockSpec((_BB, d), lambda i, j: (i, 0)),
            pl.BlockSpec((_KB, d), lambda i, j: (j, 0)),
        ],
        out_specs=pl.BlockSpec((_BB,), lambda i, j: (i,)),
        out_shape=jax.ShapeDtypeStruct((b,), jnp.int32),
        scratch_shapes=[
            pltpu.VMEM((1, _BB), jnp.float32),
            pltpu.VMEM((1, _BB), jnp.int32),
            pltpu.VMEM((1, _BB), jnp.float32),
            pltpu.VMEM((1, _BB), jnp.int32),
        ],
    )(x, table)


def _gather_rows(table, ix):
    kk, d = table.shape
    b = ix.shape[0]
    info = plsc.get_sparse_core_info()
    nw = info.num_cores * info.num_subcores
    bpw = b // nw
    ch = 128
    nch = bpw // ch
    mesh = plsc.VectorSubcoreMesh(core_axis_name="c", subcore_axis_name="s")

    @functools.partial(
        pl.kernel, mesh=mesh,
        out_type=jax.ShapeDtypeStruct((b, d), jnp.float32),
        scratch_types=[
            pltpu.VMEM((ch,), jnp.int32),
            pltpu.VMEM((ch, d), jnp.float32),
            pltpu.SemaphoreType.DMA,
        ],
    )
    def gk(table_hbm, idx_hbm, out_hbm, idx_v, rows_v, sem):
        wid = lax.axis_index("s") * info.num_cores + lax.axis_index("c")
        for c in range(nch):
            base = wid * bpw + c * ch
            pltpu.sync_copy(idx_hbm.at[pl.ds(base, ch)], idx_v)
            pltpu.async_copy(table_hbm.at[idx_v], rows_v, sem).wait()
            pltpu.sync_copy(rows_v, out_hbm.at[pl.ds(base, ch)])

    return gk(table, ix)


def kernel(x, table):
    ix = _scores_argmin(x, table)
    out = _gather_rows(table, ix)
    return out.reshape(x.shape)
